# baseline (speedup 1.0000x reference)
import functools

import jax
import jax.numpy as jnp
from jax import lax
from jax.experimental import pallas as pl
from jax.experimental.pallas import tpu as pltpu
from jax.experimental.pallas import tpu_sc as plsc

B, L, D = 4096, 200, 64
STRIDE = 256
NLANE = 16
NFULL = 12
TAIL_OFF = L - NLANE
H0, H1 = 104, 96
NW = 32
BPW = B // NW
GROUP = 32
N_CATS_P = 1010
N_PRICE = 11


def _fire(item_hbm, ids_v, r, rows_buf, sem):
    cp0 = pltpu.async_copy(
        item_hbm.at[ids_v.at[pl.ds(r * STRIDE, H0)]],
        rows_buf.at[pl.ds(0, H0)], sem)
    cp1 = pltpu.async_copy(
        item_hbm.at[ids_v.at[pl.ds(r * STRIDE + H0, H1)]],
        rows_buf.at[pl.ds(H0, H1)], sem)
    return cp0, cp1


def _wait(item_hbm, ids_v, r, rows_buf, sem):
    pltpu.make_async_copy(
        item_hbm.at[ids_v.at[pl.ds(r * STRIDE, H0)]],
        rows_buf.at[pl.ds(0, H0)], sem).wait()
    pltpu.make_async_copy(
        item_hbm.at[ids_v.at[pl.ds(r * STRIDE + H0, H1)]],
        rows_buf.at[pl.ds(H0, H1)], sem).wait()


def _compute_row(w_v, pooled_v, rows_buf, r, g):
    z = jnp.zeros((NLANE,), jnp.float32)

    def chunk_body(c, carry):
        a0, a1, a2, a3 = carry
        off = c * NLANE
        wch = w_v[pl.ds(r * STRIDE + off, NLANE)]
        for k in range(NLANE):
            wb = jnp.full((NLANE,), wch[k], jnp.float32)
            base = off + k
            a0 = a0 + wb * rows_buf[base, pl.ds(0, NLANE)]
            a1 = a1 + wb * rows_buf[base, pl.ds(NLANE, NLANE)]
            a2 = a2 + wb * rows_buf[base, pl.ds(2 * NLANE, NLANE)]
            a3 = a3 + wb * rows_buf[base, pl.ds(3 * NLANE, NLANE)]
        return (a0, a1, a2, a3)

    a0, a1, a2, a3 = lax.fori_loop(0, NFULL, chunk_body, (z, z, z, z))

    wch = w_v[pl.ds(r * STRIDE + TAIL_OFF, NLANE)]
    for k in range(8, NLANE):
        wb = jnp.full((NLANE,), wch[k], jnp.float32)
        base = TAIL_OFF + k
        a0 = a0 + wb * rows_buf[base, pl.ds(0, NLANE)]
        a1 = a1 + wb * rows_buf[base, pl.ds(NLANE, NLANE)]
        a2 = a2 + wb * rows_buf[base, pl.ds(2 * NLANE, NLANE)]
        a3 = a3 + wb * rows_buf[base, pl.ds(3 * NLANE, NLANE)]

    rg = g * GROUP + r
    pooled_v[rg, pl.ds(0, NLANE)] = a0
    pooled_v[rg, pl.ds(NLANE, NLANE)] = a1
    pooled_v[rg, pl.ds(2 * NLANE, NLANE)] = a2
    pooled_v[rg, pl.ds(3 * NLANE, NLANE)] = a3


def _sc_body(ids_hbm, w_hbm, pid_hbm, item_hbm,
             pooled_out, ipos_out,
             ids_v, w_v, rows_a, rows_b, pooled_v, pid_v, ipos_v,
             sem_a, sem_b, sem_p):
    wid = lax.axis_index("s") * 2 + lax.axis_index("c")
    base = wid * BPW

    pltpu.sync_copy(pid_hbm.at[pl.ds(base, BPW)], pid_v)
    pltpu.async_copy(item_hbm.at[pid_v], ipos_v, sem_p)

    def group_body(g, carry):
        goff = (base + g * GROUP) * STRIDE
        pltpu.sync_copy(ids_hbm.at[pl.ds(goff, GROUP * STRIDE)], ids_v)
        pltpu.sync_copy(w_hbm.at[pl.ds(goff, GROUP * STRIDE)], w_v)
        _fire(item_hbm, ids_v, 0, rows_a, sem_a)

        def pair_body(i, carry2):
            r0 = 2 * i
            _fire(item_hbm, ids_v, r0 + 1, rows_b, sem_b)
            _wait(item_hbm, ids_v, r0, rows_a, sem_a)
            _compute_row(w_v, pooled_v, rows_a, r0, g)

            @pl.when(i < GROUP // 2 - 1)
            def _():
                _fire(item_hbm, ids_v, r0 + 2, rows_a, sem_a)

            _wait(item_hbm, ids_v, r0 + 1, rows_b, sem_b)
            _compute_row(w_v, pooled_v, rows_b, r0 + 1, g)
            return carry2

        lax.fori_loop(0, GROUP // 2, pair_body, 0)
        return carry

    lax.fori_loop(0, BPW // GROUP, group_body, 0)

    pltpu.sync_copy(pooled_v, pooled_out.at[pl.ds(base, BPW)])
    pltpu.make_async_copy(item_hbm.at[pid_v], ipos_v, sem_p).wait()
    pltpu.sync_copy(ipos_v, ipos_out.at[pl.ds(base, BPW)])


_sc_pool = pl.kernel(
    _sc_body,
    out_type=(jax.ShapeDtypeStruct((B, D), jnp.float32),
              jax.ShapeDtypeStruct((B, 2 * D), jnp.float32)),
    mesh=plsc.VectorSubcoreMesh(core_axis_name="c", subcore_axis_name="s"),
    scratch_types=[
        pltpu.VMEM((GROUP * STRIDE,), jnp.int32),
        pltpu.VMEM((GROUP * STRIDE,), jnp.float32),
        pltpu.VMEM((L, 2 * D), jnp.float32),
        pltpu.VMEM((L, 2 * D), jnp.float32),
        pltpu.VMEM((BPW, D), jnp.float32),
        pltpu.VMEM((BPW,), jnp.int32),
        pltpu.VMEM((BPW, 2 * D), jnp.float32),
        pltpu.SemaphoreType.DMA,
        pltpu.SemaphoreType.DMA,
        pltpu.SemaphoreType.DMA,
    ],
    compiler_params=pltpu.CompilerParams(use_tc_tiling_on_sc=True,
                                         needs_layout_passes=False),
)


def _flat_pad(x, bm):
    pad = jnp.zeros((bm, STRIDE - L), x.dtype)
    xp = jnp.concatenate([x, pad], axis=1)
    xp = xp.reshape(bm, 2, 128)
    xp = xp.reshape(bm * 2, 128)
    return xp.reshape(bm * STRIDE)


def _prep_body(ids_ref, w_ref, idsf_ref, wmnf_ref):
    ids2 = ids_ref[...]
    w2 = w_ref[...]
    wm = jnp.where(ids2 != 0, w2, 0.0)
    s = jnp.maximum(jnp.sum(wm, axis=1, keepdims=True), 1e-8)
    bm = ids2.shape[0]
    idsf_ref[...] = _flat_pad(ids2, bm)
    wmnf_ref[...] = _flat_pad(wm / s, bm)


def _prep(ids, w):
    BM = 512
    grid = (B // BM,)
    return pl.pallas_call(
        _prep_body,
        grid=grid,
        in_specs=[pl.BlockSpec((BM, L), lambda i: (i, 0)),
                  pl.BlockSpec((BM, L), lambda i: (i, 0))],
        out_specs=[pl.BlockSpec((BM * STRIDE,), lambda i: (i,)),
                   pl.BlockSpec((BM * STRIDE,), lambda i: (i,))],
        out_shape=[jax.ShapeDtypeStruct((B * STRIDE,), jnp.int32),
                   jax.ShapeDtypeStruct((B * STRIDE,), jnp.float32)],
    )(ids, w)


def _mm(a, b):
    return lax.dot_general(a, b, (((1,), (0,)), ((), ())),
                           precision=lax.Precision.HIGHEST,
                           preferred_element_type=jnp.float32)


def _l2n(x):
    n = jnp.sqrt(jnp.sum(x * x, axis=-1, keepdims=True))
    return x / jnp.maximum(n, 1e-12)


def _tc_body(pooled, ipos, cid, prid, cat_t, price_t,
             W1a, W1b, W1c, b1, W2, b2, W3, b3,
             uW1, ub1, uW2, ub2, uW3, ub3,
             user_out, pos_out):
    h = jnp.maximum(_mm(pooled[...], uW1[...]) + ub1[...], 0.0)
    h = jnp.maximum(_mm(h, uW2[...]) + ub2[...], 0.0)
    u = _mm(h, uW3[...]) + ub3[...]
    user_out[...] = _l2n(u)

    bm = cid.shape[0]
    c_oh = (cid[...] == lax.broadcasted_iota(jnp.int32, (bm, N_CATS_P), 1))
    c_vec = _mm(c_oh.astype(jnp.float32), cat_t[...])
    p_oh = (prid[...] == lax.broadcasted_iota(jnp.int32, (bm, N_PRICE), 1))
    p_vec = _mm(p_oh.astype(jnp.float32), price_t[...])
    x1 = (_mm(ipos[...][:, :D], W1a[...]) + _mm(c_vec, W1b[...])
          + _mm(p_vec, W1c[...]) + b1[...])
    h = jnp.maximum(x1, 0.0)
    h = jnp.maximum(_mm(h, W2[...]) + b2[...], 0.0)
    v = _mm(h, W3[...]) + b3[...]
    pos_out[...] = _l2n(v)


def _tc_towers(pooled, ipos, cid, prid, cat_t, price_t,
               W1a, W1b, W1c, b1, W2, b2, W3, b3,
               uW1, ub1, uW2, ub2, uW3, ub3):
    BM = 512
    grid = (B // BM,)

    def row_spec(w):
        return pl.BlockSpec((BM, w), lambda i: (i, 0))

    def full_spec(shape):
        return pl.BlockSpec(shape, lambda i: (0,) * len(shape))

    in_specs = [
        row_spec(D), row_spec(2 * D), row_spec(1), row_spec(1),
        full_spec(cat_t.shape), full_spec(price_t.shape),
        full_spec(W1a.shape), full_spec(W1b.shape), full_spec(W1c.shape),
        full_spec(b1.shape), full_spec(W2.shape), full_spec(b2.shape),
        full_spec(W3.shape), full_spec(b3.shape),
        full_spec(uW1.shape), full_spec(ub1.shape), full_spec(uW2.shape),
        full_spec(ub2.shape), full_spec(uW3.shape), full_spec(ub3.shape),
    ]
    out_specs = [row_spec(D), row_spec(D)]
    return pl.pallas_call(
        _tc_body,
        grid=grid,
        in_specs=in_specs,
        out_specs=out_specs,
        out_shape=[jax.ShapeDtypeStruct((B, D), jnp.float32),
                   jax.ShapeDtypeStruct((B, D), jnp.float32)],
    )(pooled, ipos, cid, prid, cat_t, price_t,
      W1a, W1b, W1c, b1, W2, b2, W3, b3,
      uW1, ub1, uW2, ub2, uW3, ub3)


def kernel(history_ids, history_weights, pos_item_ids, pos_cat_ids, pos_price,
           item_emb, cat_emb, price_emb,
           it_W1, it_b1, it_W2, it_b2, it_W3, it_b3,
           us_W1, us_b1, us_W2, us_b2, us_W3, us_b3):
    ids_flat, wmn_flat = _prep(history_ids.astype(jnp.int32), history_weights)
    pid = pos_item_ids.astype(jnp.int32)

    item128 = jnp.concatenate([item_emb, jnp.zeros_like(item_emb)], axis=1)
    pooled, ipos = _sc_pool(ids_flat, wmn_flat, pid, item128)

    cid = pos_cat_ids.astype(jnp.int32).reshape(B, 1)
    prid = pos_price.astype(jnp.int32).reshape(B, 1)
    W1a, W1b, W1c = it_W1[:D], it_W1[D:D + 50], it_W1[D + 50:]
    user_emb, pos_emb = _tc_towers(
        pooled, ipos, cid, prid, cat_emb, price_emb,
        W1a, W1b, W1c, it_b1.reshape(1, -1), it_W2, it_b2.reshape(1, -1),
        it_W3, it_b3.reshape(1, -1),
        us_W1, us_b1.reshape(1, -1), us_W2, us_b2.reshape(1, -1),
        us_W3, us_b3.reshape(1, -1))
    return (user_emb, pos_emb)


# speedup vs baseline: 1.0680x; 1.0680x over previous
import functools

import jax
import jax.numpy as jnp
from jax import lax
from jax.experimental import pallas as pl
from jax.experimental.pallas import tpu as pltpu
from jax.experimental.pallas import tpu_sc as plsc

B, L, D = 4096, 200, 64
STRIDE = 256
NLANE = 16
NFULL = 12
TAIL_OFF = L - NLANE
H0, H1 = 104, 96
NW = 32
BPW = B // NW
GROUP = 32
N_CATS_P = 1010
N_PRICE = 11


def _fire(item_hbm, ids_v, r, rows_buf, sem):
    cp0 = pltpu.async_copy(
        item_hbm.at[ids_v.at[pl.ds(r * STRIDE, H0)]],
        rows_buf.at[pl.ds(0, H0)], sem)
    cp1 = pltpu.async_copy(
        item_hbm.at[ids_v.at[pl.ds(r * STRIDE + H0, H1)]],
        rows_buf.at[pl.ds(H0, H1)], sem)
    return cp0, cp1


def _wait(item_hbm, ids_v, r, rows_buf, sem):
    pltpu.make_async_copy(
        item_hbm.at[ids_v.at[pl.ds(r * STRIDE, H0)]],
        rows_buf.at[pl.ds(0, H0)], sem).wait()
    pltpu.make_async_copy(
        item_hbm.at[ids_v.at[pl.ds(r * STRIDE + H0, H1)]],
        rows_buf.at[pl.ds(H0, H1)], sem).wait()


def _compute_row(w_v, pooled_v, rows_buf, r, g):
    z = jnp.zeros((NLANE,), jnp.float32)

    def chunk_body(c, carry):
        a0, a1, a2, a3 = carry
        off = c * NLANE
        wch = w_v[pl.ds(r * STRIDE + off, NLANE)]
        for k in range(NLANE):
            wb = jnp.full((NLANE,), wch[k], jnp.float32)
            base = off + k
            a0 = a0 + wb * rows_buf[base, pl.ds(0, NLANE)]
            a1 = a1 + wb * rows_buf[base, pl.ds(NLANE, NLANE)]
            a2 = a2 + wb * rows_buf[base, pl.ds(2 * NLANE, NLANE)]
            a3 = a3 + wb * rows_buf[base, pl.ds(3 * NLANE, NLANE)]
        return (a0, a1, a2, a3)

    a0, a1, a2, a3 = lax.fori_loop(0, NFULL, chunk_body, (z, z, z, z))

    wch = w_v[pl.ds(r * STRIDE + TAIL_OFF, NLANE)]
    for k in range(8, NLANE):
        wb = jnp.full((NLANE,), wch[k], jnp.float32)
        base = TAIL_OFF + k
        a0 = a0 + wb * rows_buf[base, pl.ds(0, NLANE)]
        a1 = a1 + wb * rows_buf[base, pl.ds(NLANE, NLANE)]
        a2 = a2 + wb * rows_buf[base, pl.ds(2 * NLANE, NLANE)]
        a3 = a3 + wb * rows_buf[base, pl.ds(3 * NLANE, NLANE)]

    rg = g * GROUP + r
    pooled_v[rg, pl.ds(0, NLANE)] = a0
    pooled_v[rg, pl.ds(NLANE, NLANE)] = a1
    pooled_v[rg, pl.ds(2 * NLANE, NLANE)] = a2
    pooled_v[rg, pl.ds(3 * NLANE, NLANE)] = a3


def _sc_body(ids_hbm, w_hbm, pid_hbm, item_hbm,
             pooled_out, ipos_out,
             ids_v, w_v, rows_a, rows_b, pooled_v, pid_v, ipos_v,
             sem_a, sem_b, sem_p):
    wid = lax.axis_index("s") * 2 + lax.axis_index("c")
    base = wid * BPW

    pltpu.sync_copy(pid_hbm.at[pl.ds(base, BPW)], pid_v)
    pltpu.async_copy(item_hbm.at[pid_v], ipos_v, sem_p)

    def group_body(g, carry):
        goff = (base + g * GROUP) * STRIDE
        pltpu.sync_copy(ids_hbm.at[pl.ds(goff, GROUP * STRIDE)], ids_v)
        pltpu.sync_copy(w_hbm.at[pl.ds(goff, GROUP * STRIDE)], w_v)
        _fire(item_hbm, ids_v, 0, rows_a, sem_a)

        def pair_body(i, carry2):
            r0 = 2 * i
            _fire(item_hbm, ids_v, r0 + 1, rows_b, sem_b)
            _wait(item_hbm, ids_v, r0, rows_a, sem_a)
            _compute_row(w_v, pooled_v, rows_a, r0, g)

            @pl.when(i < GROUP // 2 - 1)
            def _():
                _fire(item_hbm, ids_v, r0 + 2, rows_a, sem_a)

            _wait(item_hbm, ids_v, r0 + 1, rows_b, sem_b)
            _compute_row(w_v, pooled_v, rows_b, r0 + 1, g)
            return carry2

        lax.fori_loop(0, GROUP // 2, pair_body, 0)
        return carry

    lax.fori_loop(0, BPW // GROUP, group_body, 0)

    pltpu.sync_copy(pooled_v, pooled_out.at[pl.ds(base, BPW)])
    pltpu.make_async_copy(item_hbm.at[pid_v], ipos_v, sem_p).wait()
    pltpu.sync_copy(ipos_v, ipos_out.at[pl.ds(base, BPW)])


_sc_pool = pl.kernel(
    _sc_body,
    out_type=(jax.ShapeDtypeStruct((B, D), jnp.float32),
              jax.ShapeDtypeStruct((B, 2 * D), jnp.float32)),
    mesh=plsc.VectorSubcoreMesh(core_axis_name="c", subcore_axis_name="s"),
    scratch_types=[
        pltpu.VMEM((GROUP * STRIDE,), jnp.int32),
        pltpu.VMEM((GROUP * STRIDE,), jnp.float32),
        pltpu.VMEM((L, 2 * D), jnp.float32),
        pltpu.VMEM((L, 2 * D), jnp.float32),
        pltpu.VMEM((BPW, D), jnp.float32),
        pltpu.VMEM((BPW,), jnp.int32),
        pltpu.VMEM((BPW, 2 * D), jnp.float32),
        pltpu.SemaphoreType.DMA,
        pltpu.SemaphoreType.DMA,
        pltpu.SemaphoreType.DMA,
    ],
    compiler_params=pltpu.CompilerParams(use_tc_tiling_on_sc=True,
                                         needs_layout_passes=False),
)


def _flat_pad(x, bm):
    pad = jnp.zeros((bm, STRIDE - L), x.dtype)
    xp = jnp.concatenate([x, pad], axis=1)
    xp = xp.reshape(bm, 2, 128)
    xp = xp.reshape(bm * 2, 128)
    return xp.reshape(bm * STRIDE)


def _prep_body(ids_ref, w_ref, idsf_ref, wmnf_ref):
    ids2 = ids_ref[...]
    w2 = w_ref[...]
    wm = jnp.where(ids2 != 0, w2, 0.0)
    s = jnp.maximum(jnp.sum(wm, axis=1, keepdims=True), 1e-8)
    bm = ids2.shape[0]
    idsf_ref[...] = _flat_pad(ids2, bm)
    wmnf_ref[...] = _flat_pad(wm / s, bm)


def _prep(ids, w):
    BM = 512
    grid = (B // BM,)
    return pl.pallas_call(
        _prep_body,
        grid=grid,
        in_specs=[pl.BlockSpec((BM, L), lambda i: (i, 0)),
                  pl.BlockSpec((BM, L), lambda i: (i, 0))],
        out_specs=[pl.BlockSpec((BM * STRIDE,), lambda i: (i,)),
                   pl.BlockSpec((BM * STRIDE,), lambda i: (i,))],
        out_shape=[jax.ShapeDtypeStruct((B * STRIDE,), jnp.int32),
                   jax.ShapeDtypeStruct((B * STRIDE,), jnp.float32)],
    )(ids, w)


def _mm(a, b):
    return lax.dot_general(a, b, (((1,), (0,)), ((), ())),
                           preferred_element_type=jnp.float32)


def _l2n(x):
    n = jnp.sqrt(jnp.sum(x * x, axis=-1, keepdims=True))
    return x / jnp.maximum(n, 1e-12)


def _tc_body(pooled, ipos, cid, prid, cat_t, price_t,
             W1a, W1b, W1c, b1, W2, b2, W3, b3,
             uW1, ub1, uW2, ub2, uW3, ub3,
             user_out, pos_out):
    h = jnp.maximum(_mm(pooled[...], uW1[...]) + ub1[...], 0.0)
    h = jnp.maximum(_mm(h, uW2[...]) + ub2[...], 0.0)
    u = _mm(h, uW3[...]) + ub3[...]
    user_out[...] = _l2n(u)

    bm = cid.shape[0]
    c_oh = (cid[...] == lax.broadcasted_iota(jnp.int32, (bm, N_CATS_P), 1))
    c_vec = _mm(c_oh.astype(jnp.float32), cat_t[...])
    p_oh = (prid[...] == lax.broadcasted_iota(jnp.int32, (bm, N_PRICE), 1))
    p_vec = _mm(p_oh.astype(jnp.float32), price_t[...])
    x1 = (_mm(ipos[...][:, :D], W1a[...]) + _mm(c_vec, W1b[...])
          + _mm(p_vec, W1c[...]) + b1[...])
    h = jnp.maximum(x1, 0.0)
    h = jnp.maximum(_mm(h, W2[...]) + b2[...], 0.0)
    v = _mm(h, W3[...]) + b3[...]
    pos_out[...] = _l2n(v)


def _tc_towers(pooled, ipos, cid, prid, cat_t, price_t,
               W1a, W1b, W1c, b1, W2, b2, W3, b3,
               uW1, ub1, uW2, ub2, uW3, ub3):
    BM = 512
    grid = (B // BM,)

    def row_spec(w):
        return pl.BlockSpec((BM, w), lambda i: (i, 0))

    def full_spec(shape):
        return pl.BlockSpec(shape, lambda i: (0,) * len(shape))

    in_specs = [
        row_spec(D), row_spec(2 * D), row_spec(1), row_spec(1),
        full_spec(cat_t.shape), full_spec(price_t.shape),
        full_spec(W1a.shape), full_spec(W1b.shape), full_spec(W1c.shape),
        full_spec(b1.shape), full_spec(W2.shape), full_spec(b2.shape),
        full_spec(W3.shape), full_spec(b3.shape),
        full_spec(uW1.shape), full_spec(ub1.shape), full_spec(uW2.shape),
        full_spec(ub2.shape), full_spec(uW3.shape), full_spec(ub3.shape),
    ]
    out_specs = [row_spec(D), row_spec(D)]
    return pl.pallas_call(
        _tc_body,
        grid=grid,
        in_specs=in_specs,
        out_specs=out_specs,
        out_shape=[jax.ShapeDtypeStruct((B, D), jnp.float32),
                   jax.ShapeDtypeStruct((B, D), jnp.float32)],
    )(pooled, ipos, cid, prid, cat_t, price_t,
      W1a, W1b, W1c, b1, W2, b2, W3, b3,
      uW1, ub1, uW2, ub2, uW3, ub3)


def kernel(history_ids, history_weights, pos_item_ids, pos_cat_ids, pos_price,
           item_emb, cat_emb, price_emb,
           it_W1, it_b1, it_W2, it_b2, it_W3, it_b3,
           us_W1, us_b1, us_W2, us_b2, us_W3, us_b3):
    ids_flat, wmn_flat = _prep(history_ids.astype(jnp.int32), history_weights)
    pid = pos_item_ids.astype(jnp.int32)

    item128 = jnp.concatenate([item_emb, jnp.zeros_like(item_emb)],
                              axis=1) + 0.0
    pooled, ipos = _sc_pool(ids_flat, wmn_flat, pid, item128)

    cid = pos_cat_ids.astype(jnp.int32).reshape(B, 1)
    prid = pos_price.astype(jnp.int32).reshape(B, 1)
    W1a, W1b, W1c = it_W1[:D], it_W1[D:D + 50], it_W1[D + 50:]
    user_emb, pos_emb = _tc_towers(
        pooled, ipos, cid, prid, cat_emb, price_emb,
        W1a, W1b, W1c, it_b1.reshape(1, -1), it_W2, it_b2.reshape(1, -1),
        it_W3, it_b3.reshape(1, -1),
        us_W1, us_b1.reshape(1, -1), us_W2, us_b2.reshape(1, -1),
        us_W3, us_b3.reshape(1, -1))
    return (user_emb, pos_emb)


# speedup vs baseline: 1.5869x; 1.4860x over previous
import functools

import jax
import jax.numpy as jnp
from jax import lax
from jax.experimental import pallas as pl
from jax.experimental.pallas import tpu as pltpu
from jax.experimental.pallas import tpu_sc as plsc

B, L, D = 4096, 200, 64
STRIDE = 256
NLANE = 16
NFULL = 12
TAIL_OFF = L - NLANE
H0, H1 = 104, 96
NW = 32
BPW = B // NW
GROUP = 32
N_CATS_P = 1010
N_PRICE = 11


def _fire(item_hbm, ids_v, r, rows_buf, sem):
    cp0 = pltpu.async_copy(
        item_hbm.at[ids_v.at[pl.ds(r * STRIDE, H0)]],
        rows_buf.at[pl.ds(0, H0)], sem)
    cp1 = pltpu.async_copy(
        item_hbm.at[ids_v.at[pl.ds(r * STRIDE + H0, H1)]],
        rows_buf.at[pl.ds(H0, H1)], sem)
    return cp0, cp1


def _wait(item_hbm, ids_v, r, rows_buf, sem):
    pltpu.make_async_copy(
        item_hbm.at[ids_v.at[pl.ds(r * STRIDE, H0)]],
        rows_buf.at[pl.ds(0, H0)], sem).wait()
    pltpu.make_async_copy(
        item_hbm.at[ids_v.at[pl.ds(r * STRIDE + H0, H1)]],
        rows_buf.at[pl.ds(H0, H1)], sem).wait()


def _compute_row(w_v, pooled_v, rows_buf, r, g):
    z = jnp.zeros((NLANE,), jnp.float32)

    def chunk_body(c, carry):
        a0, a1, a2, a3 = carry
        off = c * NLANE
        wch = w_v[pl.ds(r * STRIDE + off, NLANE)]
        for k in range(NLANE):
            wb = jnp.full((NLANE,), wch[k], jnp.float32)
            base = off + k
            a0 = a0 + wb * rows_buf[base, pl.ds(0, NLANE)]
            a1 = a1 + wb * rows_buf[base, pl.ds(NLANE, NLANE)]
            a2 = a2 + wb * rows_buf[base, pl.ds(2 * NLANE, NLANE)]
            a3 = a3 + wb * rows_buf[base, pl.ds(3 * NLANE, NLANE)]
        return (a0, a1, a2, a3)

    a0, a1, a2, a3 = lax.fori_loop(0, NFULL, chunk_body, (z, z, z, z))

    wch = w_v[pl.ds(r * STRIDE + TAIL_OFF, NLANE)]
    for k in range(8, NLANE):
        wb = jnp.full((NLANE,), wch[k], jnp.float32)
        base = TAIL_OFF + k
        a0 = a0 + wb * rows_buf[base, pl.ds(0, NLANE)]
        a1 = a1 + wb * rows_buf[base, pl.ds(NLANE, NLANE)]
        a2 = a2 + wb * rows_buf[base, pl.ds(2 * NLANE, NLANE)]
        a3 = a3 + wb * rows_buf[base, pl.ds(3 * NLANE, NLANE)]

    rg = g * GROUP + r
    pooled_v[rg, pl.ds(0, NLANE)] = a0
    pooled_v[rg, pl.ds(NLANE, NLANE)] = a1
    pooled_v[rg, pl.ds(2 * NLANE, NLANE)] = a2
    pooled_v[rg, pl.ds(3 * NLANE, NLANE)] = a3


def _sc_body(ids_hbm, w_hbm, pid_hbm, item_hbm,
             pooled_out, ipos_out,
             ids_v, w_v, rows_a, rows_b, pooled_v, pid_v, ipos_v,
             sem_a, sem_b, sem_p):
    wid = lax.axis_index("s") * 2 + lax.axis_index("c")
    base = wid * BPW

    pltpu.sync_copy(pid_hbm.at[pl.ds(base, BPW)], pid_v)
    pltpu.async_copy(item_hbm.at[pid_v], ipos_v, sem_p)

    def group_body(g, carry):
        goff = (base + g * GROUP) * STRIDE
        pltpu.sync_copy(ids_hbm.at[pl.ds(goff, GROUP * STRIDE)], ids_v)
        pltpu.sync_copy(w_hbm.at[pl.ds(goff, GROUP * STRIDE)], w_v)
        _fire(item_hbm, ids_v, 0, rows_a, sem_a)

        def pair_body(i, carry2):
            r0 = 2 * i
            _fire(item_hbm, ids_v, r0 + 1, rows_b, sem_b)
            _wait(item_hbm, ids_v, r0, rows_a, sem_a)
            _compute_row(w_v, pooled_v, rows_a, r0, g)

            @pl.when(i < GROUP // 2 - 1)
            def _():
                _fire(item_hbm, ids_v, r0 + 2, rows_a, sem_a)

            _wait(item_hbm, ids_v, r0 + 1, rows_b, sem_b)
            _compute_row(w_v, pooled_v, rows_b, r0 + 1, g)
            return carry2

        lax.fori_loop(0, GROUP // 2, pair_body, 0)
        return carry

    lax.fori_loop(0, BPW // GROUP, group_body, 0)

    pltpu.sync_copy(pooled_v, pooled_out.at[pl.ds(base, BPW)])
    pltpu.make_async_copy(item_hbm.at[pid_v], ipos_v, sem_p).wait()
    pltpu.sync_copy(ipos_v, ipos_out.at[pl.ds(base, BPW)])


_sc_pool = pl.kernel(
    _sc_body,
    out_type=(jax.ShapeDtypeStruct((B, D), jnp.float32),
              jax.ShapeDtypeStruct((B, 2 * D), jnp.float32)),
    mesh=plsc.VectorSubcoreMesh(core_axis_name="c", subcore_axis_name="s"),
    scratch_types=[
        pltpu.VMEM((GROUP * STRIDE,), jnp.int32),
        pltpu.VMEM((GROUP * STRIDE,), jnp.float32),
        pltpu.VMEM((L, 2 * D), jnp.float32),
        pltpu.VMEM((L, 2 * D), jnp.float32),
        pltpu.VMEM((BPW, D), jnp.float32),
        pltpu.VMEM((BPW,), jnp.int32),
        pltpu.VMEM((BPW, 2 * D), jnp.float32),
        pltpu.SemaphoreType.DMA,
        pltpu.SemaphoreType.DMA,
        pltpu.SemaphoreType.DMA,
    ],
    compiler_params=pltpu.CompilerParams(use_tc_tiling_on_sc=True,
                                         needs_layout_passes=False),
)


def _flat_pad(x, bm):
    pad = jnp.zeros((bm, STRIDE - L), x.dtype)
    xp = jnp.concatenate([x, pad], axis=1)
    xp = xp.reshape(bm, 2, 128)
    xp = xp.reshape(bm * 2, 128)
    return xp.reshape(bm * STRIDE)


def _prep_body(ids_ref, w_ref, idsf_ref, wmnf_ref):
    ids2 = ids_ref[...]
    w2 = w_ref[...]
    wm = jnp.where(ids2 != 0, w2, 0.0)
    s = jnp.maximum(jnp.sum(wm, axis=1, keepdims=True), 1e-8)
    bm = ids2.shape[0]
    idsf_ref[...] = _flat_pad(ids2, bm)
    wmnf_ref[...] = _flat_pad(wm / s, bm)


def _prep(ids, w):
    BM = 512
    grid = (B // BM,)
    return pl.pallas_call(
        _prep_body,
        grid=grid,
        in_specs=[pl.BlockSpec((BM, L), lambda i: (i, 0)),
                  pl.BlockSpec((BM, L), lambda i: (i, 0))],
        out_specs=[pl.BlockSpec((BM * STRIDE,), lambda i: (i,)),
                   pl.BlockSpec((BM * STRIDE,), lambda i: (i,))],
        out_shape=[jax.ShapeDtypeStruct((B * STRIDE,), jnp.int32),
                   jax.ShapeDtypeStruct((B * STRIDE,), jnp.float32)],
    )(ids, w)


PAD_BN = 8192
N_ROWS = 1000010
PAD_STEPS = -(-N_ROWS // PAD_BN)
N_ROWS_P = PAD_STEPS * PAD_BN


def _padT_body(it_ref, out_ref):
    x = it_ref[...]
    xt = x.T
    out_ref[...] = jnp.concatenate(
        [xt, jnp.zeros((PAD_BN, D), jnp.float32)], axis=1)


def _padT(item_t):
    return pl.pallas_call(
        _padT_body,
        grid=(PAD_STEPS,),
        in_specs=[pl.BlockSpec((D, PAD_BN), lambda i: (0, i))],
        out_specs=pl.BlockSpec((PAD_BN, 2 * D), lambda i: (i, 0)),
        out_shape=jax.ShapeDtypeStruct((N_ROWS_P, 2 * D), jnp.float32),
    )(item_t)


def _mm(a, b):
    return lax.dot_general(a, b, (((1,), (0,)), ((), ())),
                           preferred_element_type=jnp.float32)


def _l2n(x):
    n = jnp.sqrt(jnp.sum(x * x, axis=-1, keepdims=True))
    return x / jnp.maximum(n, 1e-12)


def _tc_body(pooled, ipos, cid, prid, cat_t, price_t,
             W1a, W1b, W1c, b1, W2, b2, W3, b3,
             uW1, ub1, uW2, ub2, uW3, ub3,
             user_out, pos_out):
    h = jnp.maximum(_mm(pooled[...], uW1[...]) + ub1[...], 0.0)
    h = jnp.maximum(_mm(h, uW2[...]) + ub2[...], 0.0)
    u = _mm(h, uW3[...]) + ub3[...]
    user_out[...] = _l2n(u)

    bm = cid.shape[0]
    c_oh = (cid[...] == lax.broadcasted_iota(jnp.int32, (bm, N_CATS_P), 1))
    c_vec = _mm(c_oh.astype(jnp.float32), cat_t[...])
    p_oh = (prid[...] == lax.broadcasted_iota(jnp.int32, (bm, N_PRICE), 1))
    p_vec = _mm(p_oh.astype(jnp.float32), price_t[...])
    x1 = (_mm(ipos[...][:, :D], W1a[...]) + _mm(c_vec, W1b[...])
          + _mm(p_vec, W1c[...]) + b1[...])
    h = jnp.maximum(x1, 0.0)
    h = jnp.maximum(_mm(h, W2[...]) + b2[...], 0.0)
    v = _mm(h, W3[...]) + b3[...]
    pos_out[...] = _l2n(v)


def _tc_towers(pooled, ipos, cid, prid, cat_t, price_t,
               W1a, W1b, W1c, b1, W2, b2, W3, b3,
               uW1, ub1, uW2, ub2, uW3, ub3):
    BM = 512
    grid = (B // BM,)

    def row_spec(w):
        return pl.BlockSpec((BM, w), lambda i: (i, 0))

    def full_spec(shape):
        return pl.BlockSpec(shape, lambda i: (0,) * len(shape))

    in_specs = [
        row_spec(D), row_spec(2 * D), row_spec(1), row_spec(1),
        full_spec(cat_t.shape), full_spec(price_t.shape),
        full_spec(W1a.shape), full_spec(W1b.shape), full_spec(W1c.shape),
        full_spec(b1.shape), full_spec(W2.shape), full_spec(b2.shape),
        full_spec(W3.shape), full_spec(b3.shape),
        full_spec(uW1.shape), full_spec(ub1.shape), full_spec(uW2.shape),
        full_spec(ub2.shape), full_spec(uW3.shape), full_spec(ub3.shape),
    ]
    out_specs = [row_spec(D), row_spec(D)]
    return pl.pallas_call(
        _tc_body,
        grid=grid,
        in_specs=in_specs,
        out_specs=out_specs,
        out_shape=[jax.ShapeDtypeStruct((B, D), jnp.float32),
                   jax.ShapeDtypeStruct((B, D), jnp.float32)],
    )(pooled, ipos, cid, prid, cat_t, price_t,
      W1a, W1b, W1c, b1, W2, b2, W3, b3,
      uW1, ub1, uW2, ub2, uW3, ub3)


def kernel(history_ids, history_weights, pos_item_ids, pos_cat_ids, pos_price,
           item_emb, cat_emb, price_emb,
           it_W1, it_b1, it_W2, it_b2, it_W3, it_b3,
           us_W1, us_b1, us_W2, us_b2, us_W3, us_b3):
    ids_flat, wmn_flat = _prep(history_ids.astype(jnp.int32), history_weights)
    pid = pos_item_ids.astype(jnp.int32)

    item128 = _padT(item_emb.T)
    pooled, ipos = _sc_pool(ids_flat, wmn_flat, pid, item128)

    cid = pos_cat_ids.astype(jnp.int32).reshape(B, 1)
    prid = pos_price.astype(jnp.int32).reshape(B, 1)
    W1a, W1b, W1c = it_W1[:D], it_W1[D:D + 50], it_W1[D + 50:]
    user_emb, pos_emb = _tc_towers(
        pooled, ipos, cid, prid, cat_emb, price_emb,
        W1a, W1b, W1c, it_b1.reshape(1, -1), it_W2, it_b2.reshape(1, -1),
        it_W3, it_b3.reshape(1, -1),
        us_W1, us_b1.reshape(1, -1), us_W2, us_b2.reshape(1, -1),
        us_W3, us_b3.reshape(1, -1))
    return (user_emb, pos_emb)


# speedup vs baseline: 1.6052x; 1.0115x over previous
import functools

import jax
import jax.numpy as jnp
from jax import lax
from jax.experimental import pallas as pl
from jax.experimental.pallas import tpu as pltpu
from jax.experimental.pallas import tpu_sc as plsc

B, L, D = 4096, 200, 64
STRIDE = 256
NLANE = 16
NFULL = 12
TAIL_OFF = L - NLANE
H0, H1 = 104, 96
NW = 32
BPW = B // NW
GROUP = 16
N_CATS_P = 1010
N_PRICE = 11


def _fire(item_hbm, ids_v, r, rows_buf, sem):
    cp0 = pltpu.async_copy(
        item_hbm.at[ids_v.at[pl.ds(r * STRIDE, H0)]],
        rows_buf.at[pl.ds(0, H0)], sem)
    cp1 = pltpu.async_copy(
        item_hbm.at[ids_v.at[pl.ds(r * STRIDE + H0, H1)]],
        rows_buf.at[pl.ds(H0, H1)], sem)
    return cp0, cp1


def _wait(item_hbm, ids_v, r, rows_buf, sem):
    pltpu.make_async_copy(
        item_hbm.at[ids_v.at[pl.ds(r * STRIDE, H0)]],
        rows_buf.at[pl.ds(0, H0)], sem).wait()
    pltpu.make_async_copy(
        item_hbm.at[ids_v.at[pl.ds(r * STRIDE + H0, H1)]],
        rows_buf.at[pl.ds(H0, H1)], sem).wait()


def _compute_row(w_v, pooled_v, rows_buf, r, g):
    z = jnp.zeros((NLANE,), jnp.float32)

    def chunk_body(c, carry):
        a0, a1, a2, a3 = carry
        off = c * NLANE
        wch = w_v[pl.ds(r * STRIDE + off, NLANE)]
        for k in range(NLANE):
            wb = jnp.full((NLANE,), wch[k], jnp.float32)
            base = off + k
            a0 = a0 + wb * rows_buf[base, pl.ds(0, NLANE)]
            a1 = a1 + wb * rows_buf[base, pl.ds(NLANE, NLANE)]
            a2 = a2 + wb * rows_buf[base, pl.ds(2 * NLANE, NLANE)]
            a3 = a3 + wb * rows_buf[base, pl.ds(3 * NLANE, NLANE)]
        return (a0, a1, a2, a3)

    a0, a1, a2, a3 = lax.fori_loop(0, NFULL, chunk_body, (z, z, z, z))

    wch = w_v[pl.ds(r * STRIDE + TAIL_OFF, NLANE)]
    for k in range(8, NLANE):
        wb = jnp.full((NLANE,), wch[k], jnp.float32)
        base = TAIL_OFF + k
        a0 = a0 + wb * rows_buf[base, pl.ds(0, NLANE)]
        a1 = a1 + wb * rows_buf[base, pl.ds(NLANE, NLANE)]
        a2 = a2 + wb * rows_buf[base, pl.ds(2 * NLANE, NLANE)]
        a3 = a3 + wb * rows_buf[base, pl.ds(3 * NLANE, NLANE)]

    rg = g * GROUP + r
    pooled_v[rg, pl.ds(0, NLANE)] = a0
    pooled_v[rg, pl.ds(NLANE, NLANE)] = a1
    pooled_v[rg, pl.ds(2 * NLANE, NLANE)] = a2
    pooled_v[rg, pl.ds(3 * NLANE, NLANE)] = a3


def _sc_body(ids_hbm, w_hbm, pid_hbm, item_hbm,
             pooled_out, ipos_out,
             ids_v, w_v, rows_a, rows_b, rows_c, rows_d, pooled_v, pid_v,
             sem_a, sem_b, sem_c, sem_d):
    wid = lax.axis_index("s") * 2 + lax.axis_index("c")
    base = wid * BPW
    bufs = ((rows_a, sem_a), (rows_b, sem_b), (rows_c, sem_c), (rows_d, sem_d))

    pltpu.sync_copy(pid_hbm.at[pl.ds(base, BPW)], pid_v)

    def group_body(g, carry):
        goff = (base + g * GROUP) * STRIDE
        pltpu.sync_copy(ids_hbm.at[pl.ds(goff, GROUP * STRIDE)], ids_v)
        pltpu.sync_copy(w_hbm.at[pl.ds(goff, GROUP * STRIDE)], w_v)
        _fire(item_hbm, ids_v, 0, rows_a, sem_a)
        _fire(item_hbm, ids_v, 1, rows_b, sem_b)

        def quad_body(i, carry2):
            r0 = 4 * i
            for j in range(4):
                buf, sem = bufs[j]
                nbuf, nsem = bufs[(j + 2) % 4]
                nr = r0 + j + 2

                @pl.when(nr < GROUP)
                def _():
                    _fire(item_hbm, ids_v, nr, nbuf, nsem)

                _wait(item_hbm, ids_v, r0 + j, buf, sem)
                _compute_row(w_v, pooled_v, buf, r0 + j, g)
            return carry2

        lax.fori_loop(0, GROUP // 4, quad_body, 0)
        return carry

    lax.fori_loop(0, BPW // GROUP, group_body, 0)

    pltpu.sync_copy(pooled_v, pooled_out.at[pl.ds(base, BPW)])
    pltpu.async_copy(item_hbm.at[pid_v], rows_a.at[pl.ds(0, BPW)],
                     sem_a).wait()
    pltpu.sync_copy(rows_a.at[pl.ds(0, BPW)], ipos_out.at[pl.ds(base, BPW)])


_sc_pool = pl.kernel(
    _sc_body,
    out_type=(jax.ShapeDtypeStruct((B, D), jnp.float32),
              jax.ShapeDtypeStruct((B, 2 * D), jnp.float32)),
    mesh=plsc.VectorSubcoreMesh(core_axis_name="c", subcore_axis_name="s"),
    scratch_types=[
        pltpu.VMEM((GROUP * STRIDE,), jnp.int32),
        pltpu.VMEM((GROUP * STRIDE,), jnp.float32),
        pltpu.VMEM((L, 2 * D), jnp.float32),
        pltpu.VMEM((L, 2 * D), jnp.float32),
        pltpu.VMEM((L, 2 * D), jnp.float32),
        pltpu.VMEM((L, 2 * D), jnp.float32),
        pltpu.VMEM((BPW, D), jnp.float32),
        pltpu.VMEM((BPW,), jnp.int32),
        pltpu.SemaphoreType.DMA,
        pltpu.SemaphoreType.DMA,
        pltpu.SemaphoreType.DMA,
        pltpu.SemaphoreType.DMA,
    ],
    compiler_params=pltpu.CompilerParams(use_tc_tiling_on_sc=True,
                                         needs_layout_passes=False),
)


def _flat_pad(x, bm):
    pad = jnp.zeros((bm, STRIDE - L), x.dtype)
    xp = jnp.concatenate([x, pad], axis=1)
    xp = xp.reshape(bm, 2, 128)
    xp = xp.reshape(bm * 2, 128)
    return xp.reshape(bm * STRIDE)


def _prep_body(ids_ref, w_ref, idsf_ref, wmnf_ref):
    ids2 = ids_ref[...]
    w2 = w_ref[...]
    wm = jnp.where(ids2 != 0, w2, 0.0)
    s = jnp.maximum(jnp.sum(wm, axis=1, keepdims=True), 1e-8)
    bm = ids2.shape[0]
    idsf_ref[...] = _flat_pad(ids2, bm)
    wmnf_ref[...] = _flat_pad(wm / s, bm)


def _prep(ids, w):
    BM = 512
    grid = (B // BM,)
    return pl.pallas_call(
        _prep_body,
        grid=grid,
        in_specs=[pl.BlockSpec((BM, L), lambda i: (i, 0)),
                  pl.BlockSpec((BM, L), lambda i: (i, 0))],
        out_specs=[pl.BlockSpec((BM * STRIDE,), lambda i: (i,)),
                   pl.BlockSpec((BM * STRIDE,), lambda i: (i,))],
        out_shape=[jax.ShapeDtypeStruct((B * STRIDE,), jnp.int32),
                   jax.ShapeDtypeStruct((B * STRIDE,), jnp.float32)],
    )(ids, w)


PAD_BN = 8192
N_ROWS = 1000010
PAD_STEPS = -(-N_ROWS // PAD_BN)
N_ROWS_P = PAD_STEPS * PAD_BN


def _padT_body(it_ref, out_ref):
    x = it_ref[...]
    xt = x.T
    out_ref[...] = jnp.concatenate(
        [xt, jnp.zeros((PAD_BN, D), jnp.float32)], axis=1)


def _padT(item_t):
    return pl.pallas_call(
        _padT_body,
        grid=(PAD_STEPS,),
        in_specs=[pl.BlockSpec((D, PAD_BN), lambda i: (0, i))],
        out_specs=pl.BlockSpec((PAD_BN, 2 * D), lambda i: (i, 0)),
        out_shape=jax.ShapeDtypeStruct((N_ROWS_P, 2 * D), jnp.float32),
    )(item_t)


def _mm(a, b):
    return lax.dot_general(a, b, (((1,), (0,)), ((), ())),
                           preferred_element_type=jnp.float32)


def _l2n(x):
    n = jnp.sqrt(jnp.sum(x * x, axis=-1, keepdims=True))
    return x / jnp.maximum(n, 1e-12)


def _tc_body(pooled, ipos, cid, prid, cat_t, price_t,
             W1a, W1b, W1c, b1, W2, b2, W3, b3,
             uW1, ub1, uW2, ub2, uW3, ub3,
             user_out, pos_out):
    h = jnp.maximum(_mm(pooled[...], uW1[...]) + ub1[...], 0.0)
    h = jnp.maximum(_mm(h, uW2[...]) + ub2[...], 0.0)
    u = _mm(h, uW3[...]) + ub3[...]
    user_out[...] = _l2n(u)

    bm = cid.shape[0]
    c_oh = (cid[...] == lax.broadcasted_iota(jnp.int32, (bm, N_CATS_P), 1))
    c_vec = _mm(c_oh.astype(jnp.float32), cat_t[...])
    p_oh = (prid[...] == lax.broadcasted_iota(jnp.int32, (bm, N_PRICE), 1))
    p_vec = _mm(p_oh.astype(jnp.float32), price_t[...])
    x1 = (_mm(ipos[...][:, :D], W1a[...]) + _mm(c_vec, W1b[...])
          + _mm(p_vec, W1c[...]) + b1[...])
    h = jnp.maximum(x1, 0.0)
    h = jnp.maximum(_mm(h, W2[...]) + b2[...], 0.0)
    v = _mm(h, W3[...]) + b3[...]
    pos_out[...] = _l2n(v)


def _tc_towers(pooled, ipos, cid, prid, cat_t, price_t,
               W1a, W1b, W1c, b1, W2, b2, W3, b3,
               uW1, ub1, uW2, ub2, uW3, ub3):
    BM = 512
    grid = (B // BM,)

    def row_spec(w):
        return pl.BlockSpec((BM, w), lambda i: (i, 0))

    def full_spec(shape):
        return pl.BlockSpec(shape, lambda i: (0,) * len(shape))

    in_specs = [
        row_spec(D), row_spec(2 * D), row_spec(1), row_spec(1),
        full_spec(cat_t.shape), full_spec(price_t.shape),
        full_spec(W1a.shape), full_spec(W1b.shape), full_spec(W1c.shape),
        full_spec(b1.shape), full_spec(W2.shape), full_spec(b2.shape),
        full_spec(W3.shape), full_spec(b3.shape),
        full_spec(uW1.shape), full_spec(ub1.shape), full_spec(uW2.shape),
        full_spec(ub2.shape), full_spec(uW3.shape), full_spec(ub3.shape),
    ]
    out_specs = [row_spec(D), row_spec(D)]
    return pl.pallas_call(
        _tc_body,
        grid=grid,
        in_specs=in_specs,
        out_specs=out_specs,
        out_shape=[jax.ShapeDtypeStruct((B, D), jnp.float32),
                   jax.ShapeDtypeStruct((B, D), jnp.float32)],
    )(pooled, ipos, cid, prid, cat_t, price_t,
      W1a, W1b, W1c, b1, W2, b2, W3, b3,
      uW1, ub1, uW2, ub2, uW3, ub3)


def kernel(history_ids, history_weights, pos_item_ids, pos_cat_ids, pos_price,
           item_emb, cat_emb, price_emb,
           it_W1, it_b1, it_W2, it_b2, it_W3, it_b3,
           us_W1, us_b1, us_W2, us_b2, us_W3, us_b3):
    ids_flat, wmn_flat = _prep(history_ids.astype(jnp.int32), history_weights)
    pid = pos_item_ids.astype(jnp.int32)

    item128 = _padT(item_emb.T)
    pooled, ipos = _sc_pool(ids_flat, wmn_flat, pid, item128)

    cid = pos_cat_ids.astype(jnp.int32).reshape(B, 1)
    prid = pos_price.astype(jnp.int32).reshape(B, 1)
    W1a, W1b, W1c = it_W1[:D], it_W1[D:D + 50], it_W1[D + 50:]
    user_emb, pos_emb = _tc_towers(
        pooled, ipos, cid, prid, cat_emb, price_emb,
        W1a, W1b, W1c, it_b1.reshape(1, -1), it_W2, it_b2.reshape(1, -1),
        it_W3, it_b3.reshape(1, -1),
        us_W1, us_b1.reshape(1, -1), us_W2, us_b2.reshape(1, -1),
        us_W3, us_b3.reshape(1, -1))
    return (user_emb, pos_emb)


# speedup vs baseline: 1.8020x; 1.1226x over previous
import functools

import jax
import jax.numpy as jnp
from jax import lax
from jax.experimental import pallas as pl
from jax.experimental.pallas import tpu as pltpu
from jax.experimental.pallas import tpu_sc as plsc

B, L, D = 4096, 200, 64
STRIDE = 256
NLANE = 16
NFULL = 12
TAIL_OFF = L - NLANE
H0, H1 = 104, 96
NW = 32
BPW = B // NW
GROUP = 16
N_CATS_P = 1010
N_PRICE = 11


def _fire(item_hbm, ids_v, r, rows_buf, sem):
    cp0 = pltpu.async_copy(
        item_hbm.at[ids_v.at[pl.ds(r * STRIDE, H0)]],
        rows_buf.at[pl.ds(0, H0)], sem)
    cp1 = pltpu.async_copy(
        item_hbm.at[ids_v.at[pl.ds(r * STRIDE + H0, H1)]],
        rows_buf.at[pl.ds(H0, H1)], sem)
    return cp0, cp1


def _wait(item_hbm, ids_v, r, rows_buf, sem):
    pltpu.make_async_copy(
        item_hbm.at[ids_v.at[pl.ds(r * STRIDE, H0)]],
        rows_buf.at[pl.ds(0, H0)], sem).wait()
    pltpu.make_async_copy(
        item_hbm.at[ids_v.at[pl.ds(r * STRIDE + H0, H1)]],
        rows_buf.at[pl.ds(H0, H1)], sem).wait()


def _compute_row(w_v, pooled_v, rows_buf, r, g):
    z = jnp.zeros((NLANE,), jnp.float32)

    def chunk_body(c, carry):
        a0, a1, a2, a3 = carry
        off = c * NLANE
        wch = w_v[pl.ds(r * STRIDE + off, NLANE)]
        for k in range(NLANE):
            wraw = wch[k]
            wp = jnp.full((NLANE,), jnp.maximum(wraw, 0.0), jnp.float32)
            wn = jnp.full((NLANE,), jnp.maximum(-wraw, 0.0), jnp.float32)
            b = off + k
            a0 = (a0 + wp * rows_buf[b, pl.ds(0, NLANE)]
                  + wn * rows_buf[b, pl.ds(4 * NLANE, NLANE)])
            a1 = (a1 + wp * rows_buf[b, pl.ds(NLANE, NLANE)]
                  + wn * rows_buf[b, pl.ds(5 * NLANE, NLANE)])
            a2 = (a2 + wp * rows_buf[b, pl.ds(2 * NLANE, NLANE)]
                  + wn * rows_buf[b, pl.ds(6 * NLANE, NLANE)])
            a3 = (a3 + wp * rows_buf[b, pl.ds(3 * NLANE, NLANE)]
                  + wn * rows_buf[b, pl.ds(7 * NLANE, NLANE)])
        return (a0, a1, a2, a3)

    a0, a1, a2, a3 = lax.fori_loop(0, NFULL, chunk_body, (z, z, z, z))

    wch = w_v[pl.ds(r * STRIDE + TAIL_OFF, NLANE)]
    for k in range(8, NLANE):
        wraw = wch[k]
        wp = jnp.full((NLANE,), jnp.maximum(wraw, 0.0), jnp.float32)
        wn = jnp.full((NLANE,), jnp.maximum(-wraw, 0.0), jnp.float32)
        b = TAIL_OFF + k
        a0 = (a0 + wp * rows_buf[b, pl.ds(0, NLANE)]
              + wn * rows_buf[b, pl.ds(4 * NLANE, NLANE)])
        a1 = (a1 + wp * rows_buf[b, pl.ds(NLANE, NLANE)]
              + wn * rows_buf[b, pl.ds(5 * NLANE, NLANE)])
        a2 = (a2 + wp * rows_buf[b, pl.ds(2 * NLANE, NLANE)]
              + wn * rows_buf[b, pl.ds(6 * NLANE, NLANE)])
        a3 = (a3 + wp * rows_buf[b, pl.ds(3 * NLANE, NLANE)]
              + wn * rows_buf[b, pl.ds(7 * NLANE, NLANE)])

    rg = g * GROUP + r
    pooled_v[rg, pl.ds(0, NLANE)] = a0
    pooled_v[rg, pl.ds(NLANE, NLANE)] = a1
    pooled_v[rg, pl.ds(2 * NLANE, NLANE)] = a2
    pooled_v[rg, pl.ds(3 * NLANE, NLANE)] = a3


def _sc_body(ids_hbm, w_hbm, pid_hbm, item_hbm,
             pooled_out, ipos_out,
             ids_v, w_v, rows_a, rows_b, rows_c, rows_d, pooled_v, pid_v,
             sem_a, sem_b, sem_c, sem_d):
    wid = lax.axis_index("s") * 2 + lax.axis_index("c")
    base = wid * BPW
    bufs = ((rows_a, sem_a), (rows_b, sem_b), (rows_c, sem_c), (rows_d, sem_d))

    pltpu.sync_copy(pid_hbm.at[pl.ds(base, BPW)], pid_v)

    def group_body(g, carry):
        goff = (base + g * GROUP) * STRIDE
        pltpu.sync_copy(ids_hbm.at[pl.ds(goff, GROUP * STRIDE)], ids_v)
        pltpu.sync_copy(w_hbm.at[pl.ds(goff, GROUP * STRIDE)], w_v)
        _fire(item_hbm, ids_v, 0, rows_a, sem_a)
        _fire(item_hbm, ids_v, 1, rows_b, sem_b)

        def quad_body(i, carry2):
            r0 = 4 * i
            for j in range(4):
                buf, sem = bufs[j]
                nbuf, nsem = bufs[(j + 2) % 4]
                nr = r0 + j + 2

                @pl.when(nr < GROUP)
                def _():
                    _fire(item_hbm, ids_v, nr, nbuf, nsem)

                _wait(item_hbm, ids_v, r0 + j, buf, sem)
                _compute_row(w_v, pooled_v, buf, r0 + j, g)
            return carry2

        lax.fori_loop(0, GROUP // 4, quad_body, 0)
        return carry

    lax.fori_loop(0, BPW // GROUP, group_body, 0)

    pltpu.sync_copy(pooled_v, pooled_out.at[pl.ds(base, BPW)])
    pltpu.async_copy(item_hbm.at[pid_v], rows_a.at[pl.ds(0, BPW)],
                     sem_a).wait()
    pltpu.sync_copy(rows_a.at[pl.ds(0, BPW)], ipos_out.at[pl.ds(base, BPW)])


_sc_pool = pl.kernel(
    _sc_body,
    out_type=(jax.ShapeDtypeStruct((B, D), jnp.float32),
              jax.ShapeDtypeStruct((B, 2 * D), jnp.float32)),
    mesh=plsc.VectorSubcoreMesh(core_axis_name="c", subcore_axis_name="s"),
    scratch_types=[
        pltpu.VMEM((GROUP * STRIDE,), jnp.int32),
        pltpu.VMEM((GROUP * STRIDE,), jnp.float32),
        pltpu.VMEM((L, 2 * D), jnp.float32),
        pltpu.VMEM((L, 2 * D), jnp.float32),
        pltpu.VMEM((L, 2 * D), jnp.float32),
        pltpu.VMEM((L, 2 * D), jnp.float32),
        pltpu.VMEM((BPW, D), jnp.float32),
        pltpu.VMEM((BPW,), jnp.int32),
        pltpu.SemaphoreType.DMA,
        pltpu.SemaphoreType.DMA,
        pltpu.SemaphoreType.DMA,
        pltpu.SemaphoreType.DMA,
    ],
    compiler_params=pltpu.CompilerParams(use_tc_tiling_on_sc=True,
                                         needs_layout_passes=False),
)


def _flat_pad(x, bm):
    pad = jnp.zeros((bm, STRIDE - L), x.dtype)
    xp = jnp.concatenate([x, pad], axis=1)
    xp = xp.reshape(bm, 2, 128)
    xp = xp.reshape(bm * 2, 128)
    return xp.reshape(bm * STRIDE)


def _prep_body(ids_ref, w_ref, idsf_ref, wmnf_ref):
    ids2 = ids_ref[...]
    w2 = w_ref[...]
    wm = jnp.where(ids2 != 0, w2, 0.0)
    s = jnp.maximum(jnp.sum(wm, axis=1, keepdims=True), 1e-8)
    bm = ids2.shape[0]
    hi = ids2 >= NH
    idsf_ref[...] = _flat_pad(jnp.where(hi, ids2 - NH, ids2), bm)
    wmnf_ref[...] = _flat_pad(jnp.where(hi, -(wm / s), wm / s), bm)


def _prep(ids, w):
    BM = 512
    grid = (B // BM,)
    return pl.pallas_call(
        _prep_body,
        grid=grid,
        in_specs=[pl.BlockSpec((BM, L), lambda i: (i, 0)),
                  pl.BlockSpec((BM, L), lambda i: (i, 0))],
        out_specs=[pl.BlockSpec((BM * STRIDE,), lambda i: (i,)),
                   pl.BlockSpec((BM * STRIDE,), lambda i: (i,))],
        out_shape=[jax.ShapeDtypeStruct((B * STRIDE,), jnp.int32),
                   jax.ShapeDtypeStruct((B * STRIDE,), jnp.float32)],
    )(ids, w)


PAD_BN = 16384
N_ROWS = 1000010
NH = 31 * PAD_BN
N_ROWS_P = 2 * NH


def _padT_body(lo_ref, hi_ref, out_ref):
    out_ref[...] = jnp.concatenate([lo_ref[...].T, hi_ref[...].T], axis=1)


def _padT(item_t):
    return pl.pallas_call(
        _padT_body,
        grid=(NH // PAD_BN,),
        in_specs=[pl.BlockSpec((D, PAD_BN), lambda i: (0, i)),
                  pl.BlockSpec((D, PAD_BN), lambda i: (0, i + NH // PAD_BN))],
        out_specs=pl.BlockSpec((PAD_BN, 2 * D), lambda i: (i, 0)),
        out_shape=jax.ShapeDtypeStruct((NH, 2 * D), jnp.float32),
    )(item_t, item_t)


def _mm(a, b):
    return lax.dot_general(a, b, (((1,), (0,)), ((), ())),
                           preferred_element_type=jnp.float32)


def _l2n(x):
    n = jnp.sqrt(jnp.sum(x * x, axis=-1, keepdims=True))
    return x / jnp.maximum(n, 1e-12)


def _tc_body(pooled, ipos, phi, cid, prid, cat_t, price_t,
             W1a, W1b, W1c, b1, W2, b2, W3, b3,
             uW1, ub1, uW2, ub2, uW3, ub3,
             user_out, pos_out):
    h = jnp.maximum(_mm(pooled[...], uW1[...]) + ub1[...], 0.0)
    h = jnp.maximum(_mm(h, uW2[...]) + ub2[...], 0.0)
    u = _mm(h, uW3[...]) + ub3[...]
    user_out[...] = _l2n(u)

    bm = cid.shape[0]
    c_oh = (cid[...] == lax.broadcasted_iota(jnp.int32, (bm, N_CATS_P), 1))
    c_vec = _mm(c_oh.astype(jnp.float32), cat_t[...])
    p_oh = (prid[...] == lax.broadcasted_iota(jnp.int32, (bm, N_PRICE), 1))
    p_vec = _mm(p_oh.astype(jnp.float32), price_t[...])
    ip = ipos[...]
    isel = jnp.where(phi[...] == 1, ip[:, D:], ip[:, :D])
    x1 = (_mm(isel, W1a[...]) + _mm(c_vec, W1b[...])
          + _mm(p_vec, W1c[...]) + b1[...])
    h = jnp.maximum(x1, 0.0)
    h = jnp.maximum(_mm(h, W2[...]) + b2[...], 0.0)
    v = _mm(h, W3[...]) + b3[...]
    pos_out[...] = _l2n(v)


def _tc_towers(pooled, ipos, phi, cid, prid, cat_t, price_t,
               W1a, W1b, W1c, b1, W2, b2, W3, b3,
               uW1, ub1, uW2, ub2, uW3, ub3):
    BM = 512
    grid = (B // BM,)

    def row_spec(w):
        return pl.BlockSpec((BM, w), lambda i: (i, 0))

    def full_spec(shape):
        return pl.BlockSpec(shape, lambda i: (0,) * len(shape))

    in_specs = [
        row_spec(D), row_spec(2 * D), row_spec(1), row_spec(1), row_spec(1),
        full_spec(cat_t.shape), full_spec(price_t.shape),
        full_spec(W1a.shape), full_spec(W1b.shape), full_spec(W1c.shape),
        full_spec(b1.shape), full_spec(W2.shape), full_spec(b2.shape),
        full_spec(W3.shape), full_spec(b3.shape),
        full_spec(uW1.shape), full_spec(ub1.shape), full_spec(uW2.shape),
        full_spec(ub2.shape), full_spec(uW3.shape), full_spec(ub3.shape),
    ]
    out_specs = [row_spec(D), row_spec(D)]
    return pl.pallas_call(
        _tc_body,
        grid=grid,
        in_specs=in_specs,
        out_specs=out_specs,
        out_shape=[jax.ShapeDtypeStruct((B, D), jnp.float32),
                   jax.ShapeDtypeStruct((B, D), jnp.float32)],
    )(pooled, ipos, phi, cid, prid, cat_t, price_t,
      W1a, W1b, W1c, b1, W2, b2, W3, b3,
      uW1, ub1, uW2, ub2, uW3, ub3)


def kernel(history_ids, history_weights, pos_item_ids, pos_cat_ids, pos_price,
           item_emb, cat_emb, price_emb,
           it_W1, it_b1, it_W2, it_b2, it_W3, it_b3,
           us_W1, us_b1, us_W2, us_b2, us_W3, us_b3):
    ids_flat, wmn_flat = _prep(history_ids.astype(jnp.int32), history_weights)
    pid = pos_item_ids.astype(jnp.int32)

    item128 = _padT(item_emb.T)
    pid_hi = (pid >= NH).astype(jnp.int32).reshape(B, 1)
    pid = jnp.where(pid >= NH, pid - NH, pid)
    pooled, ipos = _sc_pool(ids_flat, wmn_flat, pid, item128)

    cid = pos_cat_ids.astype(jnp.int32).reshape(B, 1)
    prid = pos_price.astype(jnp.int32).reshape(B, 1)
    W1a, W1b, W1c = it_W1[:D], it_W1[D:D + 50], it_W1[D + 50:]
    user_emb, pos_emb = _tc_towers(
        pooled, ipos, pid_hi, cid, prid, cat_emb, price_emb,
        W1a, W1b, W1c, it_b1.reshape(1, -1), it_W2, it_b2.reshape(1, -1),
        it_W3, it_b3.reshape(1, -1),
        us_W1, us_b1.reshape(1, -1), us_W2, us_b2.reshape(1, -1),
        us_W3, us_b3.reshape(1, -1))
    return (user_emb, pos_emb)
